# baseline (speedup 1.0000x reference)
import functools

import jax
import jax.numpy as jnp
from jax import lax
from jax.experimental import pallas as pl
from jax.experimental.pallas import tpu as pltpu
from jax.experimental.pallas import tpu_sc as plsc

N = 10000
N_PAD = 10112
D_IN = 128
D_HID = 200
N_CLASSES = 40
C_PAD = 128
E = 320000
NUM_TILES = 32
CHUNK = 128
CHUNKS = 80
E_PAD = NUM_TILES * CHUNKS * CHUNK
ROWS_PER_TILE = N_PAD // 16
NBUF = 4

_mesh = functools.partial(
    plsc.VectorSubcoreMesh, core_axis_name="c", subcore_axis_name="s")


def _deg_body(edges_hbm, out_hbm, idx_v, hist_v):
  c = lax.axis_index("c")
  s = lax.axis_index("s")
  wid = c * 16 + s
  pltpu.sync_copy(edges_hbm.at[1, wid], idx_v)
  zero16 = jnp.zeros((16,), jnp.float32)

  def zbody(i, carry):
    hist_v[pl.ds(i * 16, 16)] = zero16
    return carry

  lax.fori_loop(0, N_PAD // 16, zbody, 0)
  one16 = jnp.ones((16,), jnp.float32)

  def body(j, carry):
    for k in range(CHUNK // 16):
      idx = idx_v[j, pl.ds(k * 16, 16)]
      plsc.addupdate_scatter(hist_v, [idx], one16)
    return carry

  lax.fori_loop(0, CHUNKS, body, 0)
  pltpu.sync_copy(hist_v, out_hbm.at[c, s])


_deg_kernel = functools.partial(
    pl.kernel,
    out_type=jax.ShapeDtypeStruct((2, 16, N_PAD), jnp.float32),
    mesh=_mesh(),
    scratch_types=[
        pltpu.VMEM((CHUNKS, CHUNK), jnp.int32),
        pltpu.VMEM((N_PAD,), jnp.float32),
    ],
    compiler_params=pltpu.CompilerParams(needs_layout_passes=False),
)(_deg_body)


def _make_agg(width, nbuf, iblk):
  n_groups = CHUNKS // iblk

  def body(tab_hbm, edges_hbm, zeros_hbm, out_hbm,
           src_v, dst_v, rows_v, acc_sh, *sems):
    c = lax.axis_index("c")
    s = lax.axis_index("s")
    wid = c * 16 + s
    r0 = s * ROWS_PER_TILE

    @pl.when(c == 0)
    def _():
      pltpu.sync_copy(tab_hbm.at[pl.ds(r0, ROWS_PER_TILE)],
                      acc_sh.at[pl.ds(r0, ROWS_PER_TILE)])

    @pl.when(c != 0)
    def _():
      pltpu.sync_copy(zeros_hbm.at[pl.ds(r0, ROWS_PER_TILE)],
                      acc_sh.at[pl.ds(r0, ROWS_PER_TILE)])

    plsc.subcore_barrier()

    def group(g, carry):
      pltpu.sync_copy(edges_hbm.at[0, wid, pl.ds(g * iblk, iblk)], src_v)
      pltpu.sync_copy(edges_hbm.at[1, wid, pl.ds(g * iblk, iblk)], dst_v)
      for b in range(nbuf):
        pltpu.async_copy(tab_hbm.at[src_v.at[b]], rows_v.at[b], sems[b])

      def inner(j0, cy):
        for b in range(nbuf):
          j = j0 + b
          pltpu.make_async_copy(
              tab_hbm.at[src_v.at[j]], rows_v.at[b], sems[b]).wait()
          pltpu.sync_copy(rows_v.at[b], acc_sh.at[dst_v.at[j]], add=True)

          @pl.when(j + nbuf < iblk)
          def _():
            pltpu.async_copy(
                tab_hbm.at[src_v.at[j + nbuf]], rows_v.at[b], sems[b])
        return cy

      return lax.fori_loop(0, iblk // nbuf, lambda i, cy: inner(i * nbuf, cy),
                           carry)

    lax.fori_loop(0, n_groups, group, 0)
    plsc.subcore_barrier()
    pltpu.sync_copy(acc_sh.at[pl.ds(r0, ROWS_PER_TILE)],
                    out_hbm.at[c, pl.ds(r0, ROWS_PER_TILE)])

  return functools.partial(
      pl.kernel,
      out_type=jax.ShapeDtypeStruct((2, N_PAD, width), jnp.float32),
      mesh=_mesh(),
      scratch_types=[
          pltpu.VMEM((iblk, CHUNK), jnp.int32),
          pltpu.VMEM((iblk, CHUNK), jnp.int32),
          pltpu.VMEM((nbuf, CHUNK, width), jnp.float32),
          pltpu.VMEM_SHARED((N_PAD, width), jnp.float32),
      ] + [pltpu.SemaphoreType.DMA] * nbuf,
  )(body)


_agg128 = _make_agg(D_IN, nbuf=2, iblk=40)
_agg48 = _make_agg(C_PAD, nbuf=2, iblk=40)


def _dis_body(hists_ref, dis_ref):
  deg = jnp.sum(hists_ref[...], axis=(0, 1)) + 1.0
  dis_ref[...] = lax.rsqrt(deg)[None, :]


def _dis_call(hists):
  return pl.pallas_call(
      _dis_body,
      out_shape=jax.ShapeDtypeStruct((1, N_PAD), jnp.float32),
  )(hists)


def _scale_body(dis_ref, x_ref, ys_ref):
  ys_ref[...] = x_ref[...] * dis_ref[...]


def _scale_call(dis_col, x_pad):
  blk = N_PAD // 8
  return pl.pallas_call(
      _scale_body,
      grid=(8,),
      in_specs=[
          pl.BlockSpec((blk, 1), lambda i: (i, 0)),
          pl.BlockSpec((blk, D_IN), lambda i: (i, 0)),
      ],
      out_specs=pl.BlockSpec((blk, D_IN), lambda i: (i, 0)),
      out_shape=jax.ShapeDtypeStruct((N_PAD, D_IN), jnp.float32),
  )(dis_col, x_pad)


def _mid_body(accp_ref, dis_ref, w1_ref, b1_ref, w2_ref, ys2_ref):
  z = (accp_ref[0] + accp_ref[1]) * dis_ref[...]
  h = jnp.dot(z, w1_ref[...], preferred_element_type=jnp.float32)
  h = jnp.maximum(h + b1_ref[...], 0.0)
  g = jnp.dot(h, w2_ref[...], preferred_element_type=jnp.float32)
  ys2_ref[...] = g * dis_ref[...]


def _mid_call(accp, dis, w1, b1r, w2p):
  blk = N_PAD // 8
  return pl.pallas_call(
      _mid_body,
      grid=(8,),
      in_specs=[
          pl.BlockSpec((2, blk, D_IN), lambda i: (0, i, 0)),
          pl.BlockSpec((blk, 1), lambda i: (i, 0)),
          pl.BlockSpec((D_IN, D_HID), lambda i: (0, 0)),
          pl.BlockSpec((1, D_HID), lambda i: (0, 0)),
          pl.BlockSpec((D_HID, C_PAD), lambda i: (0, 0)),
      ],
      out_specs=pl.BlockSpec((blk, C_PAD), lambda i: (i, 0)),
      out_shape=jax.ShapeDtypeStruct((N_PAD, C_PAD), jnp.float32),
  )(accp, dis, w1, b1r, w2p)


def _out_body(agg2p_ref, dis_ref, b2_ref, out_ref):
  z = (agg2p_ref[0] + agg2p_ref[1]) * dis_ref[...] + b2_ref[...]
  z = z[:, :N_CLASSES]
  m = jnp.max(z, axis=1, keepdims=True)
  e = jnp.exp(z - m)
  lse = jnp.log(jnp.sum(e, axis=1, keepdims=True))
  out_ref[...] = z - m - lse


def _out_call(agg2p, dis, b2r):
  blk = N // 10
  return pl.pallas_call(
      _out_body,
      grid=(10,),
      in_specs=[
          pl.BlockSpec((2, blk, C_PAD), lambda i: (0, i, 0)),
          pl.BlockSpec((blk, 1), lambda i: (i, 0)),
          pl.BlockSpec((1, C_PAD), lambda i: (0, 0)),
      ],
      out_specs=pl.BlockSpec((blk, N_CLASSES), lambda i: (i, 0)),
      out_shape=jax.ShapeDtypeStruct((N, N_CLASSES), jnp.float32),
  )(agg2p, dis, b2r)


@jax.jit
def kernel(x, edge_index, W1, b1, W2, b2):
  pad = (N + jnp.arange(E_PAD - E, dtype=jnp.int32) % (N_PAD - N))
  edges_t = jnp.concatenate(
      [edge_index.astype(jnp.int32), jnp.stack([pad, pad])], axis=1
  ).reshape(2, NUM_TILES, CHUNKS, CHUNK)

  x_pad = jnp.pad(x, ((0, N_PAD - N), (0, 0)))
  z128 = jnp.zeros((N_PAD, D_IN), jnp.float32)
  w2p = jnp.pad(W2, ((0, 0), (0, C_PAD - N_CLASSES)))
  b1r = b1.reshape(1, D_HID)
  b2r = jnp.pad(b2, (0, C_PAD - N_CLASSES)).reshape(1, C_PAD)

  hists = _deg_kernel(edges_t)
  dis = _dis_call(hists).reshape(N_PAD, 1)
  ys = _scale_call(dis, x_pad)
  accp = _agg128(ys, edges_t, z128)
  ys2 = _mid_call(accp, dis, W1, b1r, w2p)
  agg2p = _agg48(ys2, edges_t, z128)
  return _out_call(agg2p, dis, b2r)


# speedup vs baseline: 1.0094x; 1.0094x over previous
import functools

import jax
import jax.numpy as jnp
from jax import lax
from jax.experimental import pallas as pl
from jax.experimental.pallas import tpu as pltpu
from jax.experimental.pallas import tpu_sc as plsc

N = 10000
N_PAD = 10112
D_IN = 128
D_HID = 200
N_CLASSES = 40
C_PAD = 128
E = 320000
NUM_TILES = 32
CHUNK = 128
CHUNKS = 80
E_PAD = NUM_TILES * CHUNKS * CHUNK
ROWS_PER_TILE = N_PAD // 16
_mesh = functools.partial(
    plsc.VectorSubcoreMesh, core_axis_name="c", subcore_axis_name="s")


def _deg_body(edges_hbm, out_hbm, idx_v, hist_v):
  c = lax.axis_index("c")
  s = lax.axis_index("s")
  wid = c * 16 + s
  pltpu.sync_copy(edges_hbm.at[1, wid], idx_v)
  zero16 = jnp.zeros((16,), jnp.float32)

  def zbody(i, carry):
    hist_v[pl.ds(i * 16, 16)] = zero16
    return carry

  lax.fori_loop(0, N_PAD // 16, zbody, 0)
  one16 = jnp.ones((16,), jnp.float32)

  def body(j, carry):
    for k in range(CHUNK // 16):
      idx = idx_v[j, pl.ds(k * 16, 16)]
      plsc.addupdate_scatter(hist_v, [idx], one16)
    return carry

  lax.fori_loop(0, CHUNKS, body, 0)
  pltpu.sync_copy(hist_v, out_hbm.at[c, s])


_deg_kernel = functools.partial(
    pl.kernel,
    out_type=jax.ShapeDtypeStruct((2, 16, N_PAD), jnp.float32),
    mesh=_mesh(),
    scratch_types=[
        pltpu.VMEM((CHUNKS, CHUNK), jnp.int32),
        pltpu.VMEM((N_PAD,), jnp.float32),
    ],
    compiler_params=pltpu.CompilerParams(needs_layout_passes=False),
)(_deg_body)


def _make_agg(width, nbuf, iblk):
  n_groups = CHUNKS // iblk

  def body(tab_hbm, edges_hbm, zeros_hbm, out_hbm,
           src_v, dst_v, rows_v, acc_sh, *sems):
    c = lax.axis_index("c")
    s = lax.axis_index("s")
    wid = c * 16 + s
    r0 = s * ROWS_PER_TILE

    @pl.when(c == 0)
    def _():
      pltpu.sync_copy(tab_hbm.at[pl.ds(r0, ROWS_PER_TILE)],
                      acc_sh.at[pl.ds(r0, ROWS_PER_TILE)])

    @pl.when(c != 0)
    def _():
      pltpu.sync_copy(zeros_hbm.at[pl.ds(r0, ROWS_PER_TILE)],
                      acc_sh.at[pl.ds(r0, ROWS_PER_TILE)])

    plsc.subcore_barrier()

    def group(g, carry):
      pltpu.sync_copy(edges_hbm.at[0, wid, pl.ds(g * iblk, iblk)], src_v)
      pltpu.sync_copy(edges_hbm.at[1, wid, pl.ds(g * iblk, iblk)], dst_v)
      for b in range(nbuf):
        pltpu.async_copy(tab_hbm.at[src_v.at[b]], rows_v.at[b], sems[b])

      def inner(j0, cy):
        for b in range(nbuf):
          j = j0 + b
          pltpu.make_async_copy(
              tab_hbm.at[src_v.at[j]], rows_v.at[b], sems[b]).wait()
          pltpu.sync_copy(rows_v.at[b], acc_sh.at[dst_v.at[j]], add=True)

          @pl.when(j + nbuf < iblk)
          def _():
            pltpu.async_copy(
                tab_hbm.at[src_v.at[j + nbuf]], rows_v.at[b], sems[b])
        return cy

      return lax.fori_loop(0, iblk // nbuf, lambda i, cy: inner(i * nbuf, cy),
                           carry)

    lax.fori_loop(0, n_groups, group, 0)
    plsc.subcore_barrier()
    pltpu.sync_copy(acc_sh.at[pl.ds(r0, ROWS_PER_TILE)],
                    out_hbm.at[c, pl.ds(r0, ROWS_PER_TILE)])

  return functools.partial(
      pl.kernel,
      out_type=jax.ShapeDtypeStruct((2, N_PAD, width), jnp.float32),
      mesh=_mesh(),
      scratch_types=[
          pltpu.VMEM((iblk, CHUNK), jnp.int32),
          pltpu.VMEM((iblk, CHUNK), jnp.int32),
          pltpu.VMEM((nbuf, CHUNK, width), jnp.float32),
          pltpu.VMEM_SHARED((N_PAD, width), jnp.float32),
      ] + [pltpu.SemaphoreType.DMA] * nbuf,
  )(body)


_agg128 = _make_agg(D_IN, nbuf=2, iblk=40)
_agg48 = _make_agg(C_PAD, nbuf=2, iblk=40)


def _dis_body(hists_ref, dis_ref):
  deg = jnp.sum(hists_ref[...], axis=(0, 1)) + 1.0
  dis_ref[...] = lax.rsqrt(deg)[None, :]


def _dis_call(hists):
  return pl.pallas_call(
      _dis_body,
      out_shape=jax.ShapeDtypeStruct((1, N_PAD), jnp.float32),
  )(hists)


def _scale_body(dis_ref, x_ref, ys_ref):
  ys_ref[...] = x_ref[...] * dis_ref[...]


def _scale_call(dis_col, x_pad):
  return pl.pallas_call(
      _scale_body,
      out_shape=jax.ShapeDtypeStruct((N_PAD, D_IN), jnp.float32),
  )(dis_col, x_pad)


def _mid_body(accp_ref, dis_ref, w1_ref, b1_ref, w2_ref, ys2_ref):
  z = (accp_ref[0] + accp_ref[1]) * dis_ref[...]
  h = jnp.dot(z, w1_ref[...], preferred_element_type=jnp.float32)
  h = jnp.maximum(h + b1_ref[...], 0.0)
  g = jnp.dot(h, w2_ref[...], preferred_element_type=jnp.float32)
  ys2_ref[...] = g * dis_ref[...]


def _mid_call(accp, dis, w1, b1r, w2p):
  return pl.pallas_call(
      _mid_body,
      out_shape=jax.ShapeDtypeStruct((N_PAD, C_PAD), jnp.float32),
  )(accp, dis, w1, b1r, w2p)


def _out_body(agg2p_ref, dis_ref, b2_ref, out_ref):
  z = (agg2p_ref[0] + agg2p_ref[1]) * dis_ref[...] + b2_ref[...]
  z = z[:N, :N_CLASSES]
  m = jnp.max(z, axis=1, keepdims=True)
  e = jnp.exp(z - m)
  lse = jnp.log(jnp.sum(e, axis=1, keepdims=True))
  out_ref[...] = z - m - lse


def _out_call(agg2p, dis, b2r):
  return pl.pallas_call(
      _out_body,
      out_shape=jax.ShapeDtypeStruct((N, N_CLASSES), jnp.float32),
  )(agg2p, dis, b2r)


@jax.jit
def kernel(x, edge_index, W1, b1, W2, b2):
  pad = (N + jnp.arange(E_PAD - E, dtype=jnp.int32) % (N_PAD - N))
  edges_t = jnp.concatenate(
      [edge_index.astype(jnp.int32), jnp.stack([pad, pad])], axis=1
  ).reshape(2, NUM_TILES, CHUNKS, CHUNK)

  x_pad = jnp.pad(x, ((0, N_PAD - N), (0, 0)))
  z128 = jnp.zeros((N_PAD, D_IN), jnp.float32)
  w2p = jnp.pad(W2, ((0, 0), (0, C_PAD - N_CLASSES)))
  b1r = b1.reshape(1, D_HID)
  b2r = jnp.pad(b2, (0, C_PAD - N_CLASSES)).reshape(1, C_PAD)

  hists = _deg_kernel(edges_t)
  dis = _dis_call(hists).reshape(N_PAD, 1)
  ys = _scale_call(dis, x_pad)
  accp = _agg128(ys, edges_t, z128)
  ys2 = _mid_call(accp, dis, W1, b1r, w2p)
  agg2p = _agg48(ys2, edges_t, z128)
  return _out_call(agg2p, dis, b2r)
